# baseline (speedup 1.0000x reference)
import functools

import jax
import jax.numpy as jnp
from jax import lax
from jax.experimental import pallas as pl
from jax.experimental.pallas import tpu as pltpu
from jax.experimental.pallas import tpu_sc as plsc

N = 10000
E = 160000
IN_DIM = 128
HID = 32

NC = 2
NS = 16
NW = NC * NS
CH = 128
KPW = 40
EPW = KPW * CH
E_PAD = NW * EPW
N_PAD = 10016
T_E = 256

_f32 = jnp.float32


@functools.cache
def _sc_mesh():
    return plsc.VectorSubcoreMesh(core_axis_name="c", subcore_axis_name="s",
                                  num_cores=NC, num_subcores=NS)


def _sc_gather_body(src_hbm, h_hbm, xg_hbm, idx_v, row_v, sem):
    wid = lax.axis_index("s") * NC + lax.axis_index("c")
    base = wid * EPW
    pltpu.sync_copy(src_hbm.at[wid], idx_v)

    def body(j, carry):
        pltpu.async_copy(h_hbm.at[idx_v.at[j]], row_v, sem).wait()
        pltpu.sync_copy(row_v, xg_hbm.at[pl.ds(base + j * CH, CH)])
        return carry

    lax.fori_loop(0, KPW, body, 0, unroll=False)


@functools.cache
def _sc_gather():
    return pl.kernel(
        _sc_gather_body,
        out_type=jax.ShapeDtypeStruct((E_PAD, HID), _f32),
        mesh=_sc_mesh(),
        compiler_params=pltpu.CompilerParams(use_tc_tiling_on_sc=False),
        scratch_types=[
            pltpu.VMEM((KPW, CH), jnp.int32),
            pltpu.VMEM((CH, HID), _f32),
            pltpu.SemaphoreType.DMA,
        ],
    )


def _make_sc_scatter(d, per_edge_vals):
    def body(dst_hbm, vals_hbm, zero_hbm, out_hbm, idx_v, val_v, acc_s, sem):
        c = lax.axis_index("c")
        s = lax.axis_index("s")
        wid = s * NC + c
        base = wid * EPW
        pltpu.sync_copy(dst_hbm.at[wid], idx_v)
        if not per_edge_vals:
            pltpu.sync_copy(vals_hbm, val_v)

        @pl.when(s == 0)
        def _():
            pltpu.sync_copy(zero_hbm, acc_s)

        plsc.subcore_barrier()

        def j_body(j, carry):
            if per_edge_vals:
                pltpu.sync_copy(vals_hbm.at[pl.ds(base + j * CH, CH)], val_v)
            pltpu.sync_copy(val_v, acc_s.at[idx_v.at[j]], add=True)
            return carry

        lax.fori_loop(0, KPW, j_body, 0, unroll=False)
        plsc.subcore_barrier()
        rpt = N_PAD // NS
        pltpu.sync_copy(acc_s.at[pl.ds(s * rpt, rpt)],
                        out_hbm.at[c].at[pl.ds(s * rpt, rpt)])

    return pl.kernel(
        body,
        out_type=jax.ShapeDtypeStruct((NC, N_PAD, d), _f32),
        mesh=_sc_mesh(),
        compiler_params=pltpu.CompilerParams(use_tc_tiling_on_sc=False),
        scratch_types=[
            pltpu.VMEM((KPW, CH), jnp.int32),
            pltpu.VMEM((CH, d), _f32),
            pltpu.VMEM_SHARED((N_PAD, d), _f32),
            pltpu.SemaphoreType.DMA,
        ],
    )


_sc_scatter_msg = functools.cache(lambda: _make_sc_scatter(HID, True))
_sc_scatter_deg = functools.cache(lambda: _make_sc_scatter(16, False))


def _lin0_body(x_ref, w_ref, b_ref, o_ref):
    o_ref[...] = jnp.maximum(
        jnp.dot(x_ref[...], w_ref[...], preferred_element_type=_f32)
        + b_ref[...], 0.0)


def _msg_body(ef_ref, xg_ref, w1_ref, b1_ref, w2_ref, b2_ref, o_ref):
    a = jnp.maximum(
        jnp.dot(ef_ref[...], w1_ref[...], preferred_element_type=_f32)
        + b1_ref[...], 0.0)
    ew = jnp.dot(a, w2_ref[...], preferred_element_type=_f32) + b2_ref[...]
    xg = xg_ref[...]
    acc = ew[:, 0:HID] * xg[:, 0:1]
    for i in range(1, HID):
        acc += ew[:, i * HID:(i + 1) * HID] * xg[:, i:i + 1]
    o_ref[...] = acc


def _gru_body(aggc_ref, degc_ref, h_ref,
              wir_ref, wiz_ref, win_ref, whr_ref, whz_ref, whn_ref,
              bir_ref, biz_ref, bin_ref, bhr_ref, bhz_ref, bhn_ref,
              bconv_ref, o_ref):
    agg = aggc_ref[0, 0:N, :] + aggc_ref[1, 0:N, :]
    deg = degc_ref[0, 0:N, 0:1] + degc_ref[1, 0:N, 0:1]
    denom = jnp.maximum(deg, 1.0)
    m = jnp.maximum(agg / denom + bconv_ref[...], 0.0)
    h = h_ref[...]

    def mm(x, w_ref):
        return jnp.dot(x, w_ref[...], preferred_element_type=_f32)

    r = jax.nn.sigmoid(mm(m, wir_ref) + bir_ref[...] + mm(h, whr_ref) + bhr_ref[...])
    z = jax.nn.sigmoid(mm(m, wiz_ref) + biz_ref[...] + mm(h, whz_ref) + bhz_ref[...])
    n = jnp.tanh(mm(m, win_ref) + bin_ref[...] + r * (mm(h, whn_ref) + bhn_ref[...]))
    o_ref[...] = (1.0 - z) * n + z * h


def _s2s_body(x_ref,
              wqi_ref, wqf_ref, wqg_ref, wqo_ref,
              whi_ref, whf_ref, whg_ref, who_ref,
              bi_ref, bf_ref, bg_ref, bo_ref, o_ref):
    x = x_ref[...]
    hl = jnp.zeros((1, HID), _f32)
    cl = jnp.zeros((1, HID), _f32)
    q = jnp.zeros((1, 2 * HID), _f32)

    def mm(v, w_ref):
        return jnp.dot(v, w_ref[...], preferred_element_type=_f32)

    for _ in range(3):
        ig = jax.nn.sigmoid(mm(q, wqi_ref) + mm(hl, whi_ref) + bi_ref[...])
        fg = jax.nn.sigmoid(mm(q, wqf_ref) + mm(hl, whf_ref) + bf_ref[...])
        gg = jnp.tanh(mm(q, wqg_ref) + mm(hl, whg_ref) + bg_ref[...])
        og = jax.nn.sigmoid(mm(q, wqo_ref) + mm(hl, who_ref) + bo_ref[...])
        cl = fg * cl + ig * gg
        hl = og * jnp.tanh(cl)
        e = jnp.sum(x * hl, axis=1, keepdims=True)
        mx = jnp.max(e, axis=0, keepdims=True)
        ex = jnp.exp(e - mx)
        alpha = ex / jnp.sum(ex, axis=0, keepdims=True)
        ro = jnp.sum(alpha * x, axis=0, keepdims=True)
        q = jnp.concatenate([hl, ro], axis=1)
    o_ref[...] = q


def _whole(shape):
    return pl.BlockSpec(shape, lambda *_: tuple(0 for _ in shape))


def _tc_lin0(nfeat, w0t, b0r):
    return pl.pallas_call(
        _lin0_body,
        out_shape=jax.ShapeDtypeStruct((N, HID), _f32),
        in_specs=[_whole((N, IN_DIM)), _whole((IN_DIM, HID)), _whole((1, HID))],
        out_specs=_whole((N, HID)),
    )(nfeat, w0t, b0r)


def _tc_msg(ef_pad, xg, w1t, b1r, w2t, b2r):
    grid = (E_PAD // T_E,)
    return pl.pallas_call(
        _msg_body,
        grid=grid,
        in_specs=[
            pl.BlockSpec((T_E, 8), lambda t: (t, 0)),
            pl.BlockSpec((T_E, HID), lambda t: (t, 0)),
            pl.BlockSpec((8, IN_DIM), lambda t: (0, 0)),
            pl.BlockSpec((1, IN_DIM), lambda t: (0, 0)),
            pl.BlockSpec((IN_DIM, HID * HID), lambda t: (0, 0)),
            pl.BlockSpec((1, HID * HID), lambda t: (0, 0)),
        ],
        out_specs=pl.BlockSpec((T_E, HID), lambda t: (t, 0)),
        out_shape=jax.ShapeDtypeStruct((E_PAD, HID), _f32),
    )(ef_pad, xg, w1t, b1r, w2t, b2r)


def _tc_gru(aggc, degc, h, weights, biases, bconv):
    specs = ([_whole((NC, N_PAD, HID)), _whole((NC, N_PAD, 16)),
              _whole((N, HID))]
             + [_whole((HID, HID))] * 6
             + [_whole((1, HID))] * 7)
    return pl.pallas_call(
        _gru_body,
        out_shape=jax.ShapeDtypeStruct((N, HID), _f32),
        in_specs=specs,
        out_specs=_whole((N, HID)),
    )(aggc, degc, h, *weights, *biases, bconv)


def _tc_s2s(x, wq, wh, b):
    specs = ([_whole((N, HID))]
             + [_whole((2 * HID, HID))] * 4
             + [_whole((HID, HID))] * 4
             + [_whole((1, HID))] * 4)
    return pl.pallas_call(
        _s2s_body,
        out_shape=jax.ShapeDtypeStruct((1, 2 * HID), _f32),
        in_specs=specs,
        out_specs=_whole((1, 2 * HID)),
    )(x, *wq, *wh, *b)


def kernel(nfeat, efeat, edge_index, W0, b0, Wb1, bb1, Wb2, bb2, b_conv,
           gru_Wih, gru_Whh, gru_bih, gru_bhh,
           s2s_Wih, s2s_Whh, s2s_bih, s2s_bhh):
    src = edge_index[0]
    dst = edge_index[1]
    pad = E_PAD - E
    src_p = jnp.concatenate([src, jnp.zeros((pad,), jnp.int32)]
                            ).reshape(NW, KPW, CH)
    dst_p = jnp.concatenate([dst, jnp.full((pad,), N, jnp.int32)]
                            ).reshape(NW, KPW, CH)
    ef_pad = jnp.zeros((E_PAD, 8), _f32).at[:E, :5].set(efeat)

    zeros32 = jnp.zeros((N_PAD, HID), _f32)
    zeros16 = jnp.zeros((N_PAD, 16), _f32)
    ones16 = jnp.ones((CH, 16), _f32)

    w0t = W0.T
    b0r = b0[None, :]
    w1t = jnp.zeros((8, IN_DIM), _f32).at[:5, :].set(Wb1.T)
    b1r = bb1[None, :]
    w2t = Wb2.T
    b2r = bb2[None, :]
    g_w = tuple(gru_Wih[i * HID:(i + 1) * HID, :].T for i in range(3)) + \
          tuple(gru_Whh[i * HID:(i + 1) * HID, :].T for i in range(3))
    g_b = tuple(gru_bih[i * HID:(i + 1) * HID][None, :] for i in range(3)) + \
          tuple(gru_bhh[i * HID:(i + 1) * HID][None, :] for i in range(3))
    s_wq = tuple(s2s_Wih[i * HID:(i + 1) * HID, :].T for i in range(4))
    s_wh = tuple(s2s_Whh[i * HID:(i + 1) * HID, :].T for i in range(4))
    s_b = tuple((s2s_bih + s2s_bhh)[i * HID:(i + 1) * HID][None, :]
                for i in range(4))

    h = _tc_lin0(nfeat, w0t, b0r)
    degc = _sc_scatter_deg()(dst_p, ones16, zeros16)
    for _ in range(3):
        xg = _sc_gather()(src_p, h)
        msg = _tc_msg(ef_pad, xg, w1t, b1r, w2t, b2r)
        aggc = _sc_scatter_msg()(dst_p, msg, zeros32)
        h = _tc_gru(aggc, degc, h, g_w, g_b, b_conv[None, :])
    q_star = _tc_s2s(h, s_wq, s_wh, s_b)
    return (q_star, h)


# speedup vs baseline: 1.7834x; 1.7834x over previous
import functools

import jax
import jax.numpy as jnp
from jax import lax
from jax.experimental import pallas as pl
from jax.experimental.pallas import tpu as pltpu
from jax.experimental.pallas import tpu_sc as plsc

N = 10000
E = 160000
IN_DIM = 128
HID = 32

NC = 2
NS = 16
NW = NC * NS
CH = 128
KPW = 40
EPW = KPW * CH
E_PAD = NW * EPW
N_PAD = 10112
T_E = 256

_f32 = jnp.float32


@functools.cache
def _sc_mesh():
    return plsc.VectorSubcoreMesh(core_axis_name="c", subcore_axis_name="s",
                                  num_cores=NC, num_subcores=NS)


GRP = 4


def _sc_gather_body(src_hbm, h_hbm, xg_hbm, idx_v, row_v, sem):
    wid = lax.axis_index("s") * NC + lax.axis_index("c")
    base = wid * EPW
    pltpu.sync_copy(src_hbm.at[wid], idx_v)

    def body(g, carry):
        j0 = g * GRP
        cps = [pltpu.async_copy(h_hbm.at[idx_v.at[j0 + b]], row_v.at[b], sem)
               for b in range(GRP)]
        for b in range(GRP):
            cps[b].wait()
            pltpu.sync_copy(row_v.at[b],
                            xg_hbm.at[pl.ds(base + (j0 + b) * CH, CH)])
        return carry

    lax.fori_loop(0, KPW // GRP, body, 0, unroll=False)


@functools.cache
def _sc_gather():
    return pl.kernel(
        _sc_gather_body,
        out_type=jax.ShapeDtypeStruct((E_PAD, HID), _f32),
        mesh=_sc_mesh(),
        compiler_params=pltpu.CompilerParams(use_tc_tiling_on_sc=False),
        scratch_types=[
            pltpu.VMEM((KPW, CH), jnp.int32),
            pltpu.VMEM((GRP, CH, HID), _f32),
            pltpu.SemaphoreType.DMA,
        ],
    )


def _make_sc_scatter(d, per_edge_vals):
    def body(dst_hbm, vals_hbm, zero_hbm, out_hbm, idx_v, val_v, acc_s, sem,
             sem2):
        c = lax.axis_index("c")
        s = lax.axis_index("s")
        wid = s * NC + c
        base = wid * EPW
        pltpu.sync_copy(dst_hbm.at[wid], idx_v)
        if not per_edge_vals:
            for b in range(GRP):
                pltpu.sync_copy(vals_hbm, val_v.at[b])

        @pl.when(s == 0)
        def _():
            pltpu.sync_copy(zero_hbm, acc_s)

        plsc.subcore_barrier()

        def j_body(g, carry):
            j0 = g * GRP
            if per_edge_vals:
                lds = [pltpu.async_copy(
                    vals_hbm.at[pl.ds(base + (j0 + b) * CH, CH)],
                    val_v.at[b], sem) for b in range(GRP)]
                scs = []
                for b in range(GRP):
                    lds[b].wait()
                    scs.append(pltpu.async_copy(
                        val_v.at[b], acc_s.at[idx_v.at[j0 + b]], sem2,
                        add=True))
                for cp in scs:
                    cp.wait()
            else:
                for b in range(GRP):
                    pltpu.sync_copy(val_v.at[b], acc_s.at[idx_v.at[j0 + b]],
                                    add=True)
            return carry

        lax.fori_loop(0, KPW // GRP, j_body, 0, unroll=False)
        plsc.subcore_barrier()
        rpt = N_PAD // NS
        pltpu.sync_copy(acc_s.at[pl.ds(s * rpt, rpt)],
                        out_hbm.at[c].at[pl.ds(s * rpt, rpt)])

    return pl.kernel(
        body,
        out_type=jax.ShapeDtypeStruct((NC, N_PAD, d), _f32),
        mesh=_sc_mesh(),
        compiler_params=pltpu.CompilerParams(use_tc_tiling_on_sc=False),
        scratch_types=[
            pltpu.VMEM((KPW, CH), jnp.int32),
            pltpu.VMEM((GRP, CH, d), _f32),
            pltpu.VMEM_SHARED((N_PAD, d), _f32),
            pltpu.SemaphoreType.DMA,
            pltpu.SemaphoreType.DMA,
        ],
    )


_sc_scatter_msg = functools.cache(lambda: _make_sc_scatter(HID, True))
_sc_scatter_deg = functools.cache(lambda: _make_sc_scatter(16, False))


def _lin0_body(x_ref, w_ref, b_ref, o_ref):
    o_ref[...] = jnp.maximum(
        jnp.dot(x_ref[...], w_ref[...], preferred_element_type=_f32)
        + b_ref[...], 0.0)


def _msg_body(ef_ref, xg_ref, w1_ref, b1_ref, w2_ref, b2_ref, o_ref):
    a = jnp.maximum(
        jnp.dot(ef_ref[...], w1_ref[...], preferred_element_type=_f32)
        + b1_ref[...], 0.0)
    ew = jnp.dot(a.astype(jnp.bfloat16), w2_ref[...],
                 preferred_element_type=_f32) + b2_ref[...]
    xg = xg_ref[...]
    acc = None
    for g in range(HID // 4):
        xgx = jnp.concatenate(
            [jnp.broadcast_to(xg[:, 4 * g + p:4 * g + p + 1], (T_E, HID))
             for p in range(4)], axis=1)
        term = ew[:, 128 * g:128 * (g + 1)] * xgx
        acc = term if acc is None else acc + term
    o_ref[...] = (acc[:, 0:HID] + acc[:, HID:2 * HID]
                  + acc[:, 2 * HID:3 * HID] + acc[:, 3 * HID:4 * HID])


def _gru_body(aggc_ref, degc_ref, h_ref,
              wir_ref, wiz_ref, win_ref, whr_ref, whz_ref, whn_ref,
              bir_ref, biz_ref, bin_ref, bhr_ref, bhz_ref, bhn_ref,
              bconv_ref, o_ref):
    agg = aggc_ref[0, 0:N, :] + aggc_ref[1, 0:N, :]
    deg = degc_ref[0, 0:N, 0:1] + degc_ref[1, 0:N, 0:1]
    denom = jnp.maximum(deg, 1.0)
    m = jnp.maximum(agg / denom + bconv_ref[...], 0.0)
    h = h_ref[...]

    def mm(x, w_ref):
        return jnp.dot(x, w_ref[...], preferred_element_type=_f32)

    r = jax.nn.sigmoid(mm(m, wir_ref) + bir_ref[...] + mm(h, whr_ref) + bhr_ref[...])
    z = jax.nn.sigmoid(mm(m, wiz_ref) + biz_ref[...] + mm(h, whz_ref) + bhz_ref[...])
    n = jnp.tanh(mm(m, win_ref) + bin_ref[...] + r * (mm(h, whn_ref) + bhn_ref[...]))
    o_ref[...] = (1.0 - z) * n + z * h


def _s2s_body(x_ref,
              wqi_ref, wqf_ref, wqg_ref, wqo_ref,
              whi_ref, whf_ref, whg_ref, who_ref,
              bi_ref, bf_ref, bg_ref, bo_ref, o_ref):
    x = x_ref[...]
    hl = jnp.zeros((1, HID), _f32)
    cl = jnp.zeros((1, HID), _f32)
    q = jnp.zeros((1, 2 * HID), _f32)

    def mm(v, w_ref):
        return jnp.dot(v, w_ref[...], preferred_element_type=_f32)

    for _ in range(3):
        ig = jax.nn.sigmoid(mm(q, wqi_ref) + mm(hl, whi_ref) + bi_ref[...])
        fg = jax.nn.sigmoid(mm(q, wqf_ref) + mm(hl, whf_ref) + bf_ref[...])
        gg = jnp.tanh(mm(q, wqg_ref) + mm(hl, whg_ref) + bg_ref[...])
        og = jax.nn.sigmoid(mm(q, wqo_ref) + mm(hl, who_ref) + bo_ref[...])
        cl = fg * cl + ig * gg
        hl = og * jnp.tanh(cl)
        e = jnp.sum(x * hl, axis=1, keepdims=True)
        mx = jnp.max(e, axis=0, keepdims=True)
        ex = jnp.exp(e - mx)
        alpha = ex / jnp.sum(ex, axis=0, keepdims=True)
        ro = jnp.sum(alpha * x, axis=0, keepdims=True)
        q = jnp.concatenate([hl, ro], axis=1)
    o_ref[...] = q


def _whole(shape):
    return pl.BlockSpec(shape, lambda *_: tuple(0 for _ in shape))


def _tc_lin0(nfeat, w0t, b0r):
    return pl.pallas_call(
        _lin0_body,
        out_shape=jax.ShapeDtypeStruct((N, HID), _f32),
        in_specs=[_whole((N, IN_DIM)), _whole((IN_DIM, HID)), _whole((1, HID))],
        out_specs=_whole((N, HID)),
    )(nfeat, w0t, b0r)


def _tc_msg(ef_pad, xg, w1t, b1r, w2t, b2r):
    grid = (E_PAD // T_E,)
    return pl.pallas_call(
        _msg_body,
        grid=grid,
        in_specs=[
            pl.BlockSpec((T_E, 8), lambda t: (t, 0)),
            pl.BlockSpec((T_E, HID), lambda t: (t, 0)),
            pl.BlockSpec((8, IN_DIM), lambda t: (0, 0)),
            pl.BlockSpec((1, IN_DIM), lambda t: (0, 0)),
            pl.BlockSpec((IN_DIM, HID * HID), lambda t: (0, 0)),

            pl.BlockSpec((1, HID * HID), lambda t: (0, 0)),
        ],
        out_specs=pl.BlockSpec((T_E, HID), lambda t: (t, 0)),
        out_shape=jax.ShapeDtypeStruct((E_PAD, HID), _f32),
    )(ef_pad, xg, w1t, b1r, w2t, b2r)


def _tc_gru(aggc, degc, h, weights, biases, bconv):
    specs = ([_whole((NC, N_PAD, HID)), _whole((NC, N_PAD, 16)),
              _whole((N, HID))]
             + [_whole((HID, HID))] * 6
             + [_whole((1, HID))] * 7)
    return pl.pallas_call(
        _gru_body,
        out_shape=jax.ShapeDtypeStruct((N, HID), _f32),
        in_specs=specs,
        out_specs=_whole((N, HID)),
    )(aggc, degc, h, *weights, *biases, bconv)


def _tc_s2s(x, wq, wh, b):
    specs = ([_whole((N, HID))]
             + [_whole((2 * HID, HID))] * 4
             + [_whole((HID, HID))] * 4
             + [_whole((1, HID))] * 4)
    return pl.pallas_call(
        _s2s_body,
        out_shape=jax.ShapeDtypeStruct((1, 2 * HID), _f32),
        in_specs=specs,
        out_specs=_whole((1, 2 * HID)),
    )(x, *wq, *wh, *b)


def kernel(nfeat, efeat, edge_index, W0, b0, Wb1, bb1, Wb2, bb2, b_conv,
           gru_Wih, gru_Whh, gru_bih, gru_bhh,
           s2s_Wih, s2s_Whh, s2s_bih, s2s_bhh):
    src = edge_index[0]
    dst = edge_index[1]
    pad = E_PAD - E
    src_p = jnp.concatenate([src, jnp.zeros((pad,), jnp.int32)]
                            ).reshape(NW, KPW, CH)
    dst_p = jnp.concatenate([dst, jnp.full((pad,), N, jnp.int32)]
                            ).reshape(NW, KPW, CH)
    ef_pad = jnp.zeros((E_PAD, 8), _f32).at[:E, :5].set(efeat)

    zeros32 = jnp.zeros((N_PAD, HID), _f32)
    zeros16 = jnp.zeros((N_PAD, 16), _f32)
    ones16 = jnp.ones((CH, 16), _f32)

    w0t = W0.T
    b0r = b0[None, :]
    w1t = jnp.zeros((8, IN_DIM), _f32).at[:5, :].set(Wb1.T)
    b1r = bb1[None, :]
    w2t = Wb2.T.astype(jnp.bfloat16)
    b2r = bb2[None, :]
    g_w = tuple(gru_Wih[i * HID:(i + 1) * HID, :].T for i in range(3)) + \
          tuple(gru_Whh[i * HID:(i + 1) * HID, :].T for i in range(3))
    g_b = tuple(gru_bih[i * HID:(i + 1) * HID][None, :] for i in range(3)) + \
          tuple(gru_bhh[i * HID:(i + 1) * HID][None, :] for i in range(3))
    s_wq = tuple(s2s_Wih[i * HID:(i + 1) * HID, :].T for i in range(4))
    s_wh = tuple(s2s_Whh[i * HID:(i + 1) * HID, :].T for i in range(4))
    s_b = tuple((s2s_bih + s2s_bhh)[i * HID:(i + 1) * HID][None, :]
                for i in range(4))

    h = _tc_lin0(nfeat, w0t, b0r)
    degc = _sc_scatter_deg()(dst_p, ones16, zeros16)
    for _ in range(3):
        xg = _sc_gather()(src_p, h)
        msg = _tc_msg(ef_pad, xg, w1t, b1r, w2t, b2r)
        aggc = _sc_scatter_msg()(dst_p, msg, zeros32)
        h = _tc_gru(aggc, degc, h, g_w, g_b, b_conv[None, :])
    q_star = _tc_s2s(h, s_wq, s_wh, s_b)
    return (q_star, h)


# speedup vs baseline: 3.9165x; 2.1961x over previous
import functools

import jax
import jax.numpy as jnp
from jax import lax
from jax.experimental import pallas as pl
from jax.experimental.pallas import tpu as pltpu
from jax.experimental.pallas import tpu_sc as plsc

N = 10000
E = 160000
IN_DIM = 128
HID = 32

NC = 2
NS = 16
NW = NC * NS
CH = 128
KPW = 40
EPW = KPW * CH
E_PAD = NW * EPW
N_PAD = 10112
T_E = 512

_f32 = jnp.float32


@functools.cache
def _sc_mesh():
    return plsc.VectorSubcoreMesh(core_axis_name="c", subcore_axis_name="s",
                                  num_cores=NC, num_subcores=NS)


GRP = 4


def _sc_gather_body(src_hbm, h_hbm, xg_hbm, idx_v, row_v, sem):
    wid = lax.axis_index("s") * NC + lax.axis_index("c")
    base = wid * EPW
    pltpu.sync_copy(src_hbm.at[wid], idx_v)

    def body(g, carry):
        j0 = g * GRP
        cps = [pltpu.async_copy(h_hbm.at[idx_v.at[j0 + b]], row_v.at[b], sem)
               for b in range(GRP)]
        for b in range(GRP):
            cps[b].wait()
            pltpu.sync_copy(row_v.at[b],
                            xg_hbm.at[pl.ds(base + (j0 + b) * CH, CH)])
        return carry

    lax.fori_loop(0, KPW // GRP, body, 0, unroll=False)


@functools.cache
def _sc_gather():
    return pl.kernel(
        _sc_gather_body,
        out_type=jax.ShapeDtypeStruct((E_PAD, HID), _f32),
        mesh=_sc_mesh(),
        compiler_params=pltpu.CompilerParams(use_tc_tiling_on_sc=False),
        scratch_types=[
            pltpu.VMEM((KPW, CH), jnp.int32),
            pltpu.VMEM((GRP, CH, HID), _f32),
            pltpu.SemaphoreType.DMA,
        ],
    )


def _make_sc_scatter(d, per_edge_vals):
    def body(dst_hbm, vals_hbm, zero_hbm, out_hbm, idx_v, val_v, acc_s, sem,
             sem2):
        c = lax.axis_index("c")
        s = lax.axis_index("s")
        wid = s * NC + c
        base = wid * EPW
        pltpu.sync_copy(dst_hbm.at[wid], idx_v)
        if not per_edge_vals:
            for b in range(GRP):
                pltpu.sync_copy(vals_hbm, val_v.at[b])

        @pl.when(s == 0)
        def _():
            pltpu.sync_copy(zero_hbm, acc_s)

        plsc.subcore_barrier()

        def j_body(g, carry):
            j0 = g * GRP
            if per_edge_vals:
                lds = [pltpu.async_copy(
                    vals_hbm.at[pl.ds(base + (j0 + b) * CH, CH)],
                    val_v.at[b], sem) for b in range(GRP)]
                scs = []
                for b in range(GRP):
                    lds[b].wait()
                    scs.append(pltpu.async_copy(
                        val_v.at[b], acc_s.at[idx_v.at[j0 + b]], sem2,
                        add=True))
                for cp in scs:
                    cp.wait()
            else:
                for b in range(GRP):
                    pltpu.sync_copy(val_v.at[b], acc_s.at[idx_v.at[j0 + b]],
                                    add=True)
            return carry

        lax.fori_loop(0, KPW // GRP, j_body, 0, unroll=False)
        plsc.subcore_barrier()
        rpt = N_PAD // NS
        pltpu.sync_copy(acc_s.at[pl.ds(s * rpt, rpt)],
                        out_hbm.at[c].at[pl.ds(s * rpt, rpt)])

    return pl.kernel(
        body,
        out_type=jax.ShapeDtypeStruct((NC, N_PAD, d), _f32),
        mesh=_sc_mesh(),
        compiler_params=pltpu.CompilerParams(use_tc_tiling_on_sc=False),
        scratch_types=[
            pltpu.VMEM((KPW, CH), jnp.int32),
            pltpu.VMEM((GRP, CH, d), _f32),
            pltpu.VMEM_SHARED((N_PAD, d), _f32),
            pltpu.SemaphoreType.DMA,
            pltpu.SemaphoreType.DMA,
        ],
    )


_sc_scatter_msg = functools.cache(lambda: _make_sc_scatter(HID, True))
_sc_scatter_deg = functools.cache(lambda: _make_sc_scatter(16, False))


def _lin0_body(x_ref, w_ref, b_ref, o_ref):
    o_ref[...] = jnp.maximum(
        jnp.dot(x_ref[...], w_ref[...], preferred_element_type=_f32)
        + b_ref[...], 0.0)


def _msg_body(ef_ref, xg_ref, w1_ref, b1_ref, w2_ref, b2_ref, o_ref):
    aT = jnp.maximum(
        jnp.dot(w1_ref[...], ef_ref[...], preferred_element_type=_f32)
        + b1_ref[...], 0.0)
    ewT = jnp.dot(w2_ref[...], aT.astype(jnp.bfloat16),
                  preferred_element_type=_f32) + b2_ref[...]
    xgT = jnp.transpose(xg_ref[...])
    acc = None
    for i in range(HID):
        term = ewT[HID * i:HID * (i + 1), :] * xgT[i:i + 1, :]
        acc = term if acc is None else acc + term
    o_ref[...] = jnp.transpose(acc)


def _gru_body(aggc_ref, degc_ref, h_ref,
              wir_ref, wiz_ref, win_ref, whr_ref, whz_ref, whn_ref,
              bir_ref, biz_ref, bin_ref, bhr_ref, bhz_ref, bhn_ref,
              bconv_ref, o_ref):
    agg = aggc_ref[0, 0:N, :] + aggc_ref[1, 0:N, :]
    deg = degc_ref[0, 0:N, 0:1] + degc_ref[1, 0:N, 0:1]
    denom = jnp.maximum(deg, 1.0)
    m = jnp.maximum(agg / denom + bconv_ref[...], 0.0)
    h = h_ref[...]

    def mm(x, w_ref):
        return jnp.dot(x, w_ref[...], preferred_element_type=_f32)

    r = jax.nn.sigmoid(mm(m, wir_ref) + bir_ref[...] + mm(h, whr_ref) + bhr_ref[...])
    z = jax.nn.sigmoid(mm(m, wiz_ref) + biz_ref[...] + mm(h, whz_ref) + bhz_ref[...])
    n = jnp.tanh(mm(m, win_ref) + bin_ref[...] + r * (mm(h, whn_ref) + bhn_ref[...]))
    o_ref[...] = (1.0 - z) * n + z * h


def _s2s_body(x_ref,
              wqi_ref, wqf_ref, wqg_ref, wqo_ref,
              whi_ref, whf_ref, whg_ref, who_ref,
              bi_ref, bf_ref, bg_ref, bo_ref, o_ref):
    x = x_ref[...]
    hl = jnp.zeros((1, HID), _f32)
    cl = jnp.zeros((1, HID), _f32)
    q = jnp.zeros((1, 2 * HID), _f32)

    def mm(v, w_ref):
        return jnp.dot(v, w_ref[...], preferred_element_type=_f32)

    for _ in range(3):
        ig = jax.nn.sigmoid(mm(q, wqi_ref) + mm(hl, whi_ref) + bi_ref[...])
        fg = jax.nn.sigmoid(mm(q, wqf_ref) + mm(hl, whf_ref) + bf_ref[...])
        gg = jnp.tanh(mm(q, wqg_ref) + mm(hl, whg_ref) + bg_ref[...])
        og = jax.nn.sigmoid(mm(q, wqo_ref) + mm(hl, who_ref) + bo_ref[...])
        cl = fg * cl + ig * gg
        hl = og * jnp.tanh(cl)
        e = jnp.sum(x * hl, axis=1, keepdims=True)
        mx = jnp.max(e, axis=0, keepdims=True)
        ex = jnp.exp(e - mx)
        alpha = ex / jnp.sum(ex, axis=0, keepdims=True)
        ro = jnp.sum(alpha * x, axis=0, keepdims=True)
        q = jnp.concatenate([hl, ro], axis=1)
    o_ref[...] = q


def _whole(shape):
    return pl.BlockSpec(shape, lambda *_: tuple(0 for _ in shape))


def _tc_lin0(nfeat, w0t, b0r):
    return pl.pallas_call(
        _lin0_body,
        out_shape=jax.ShapeDtypeStruct((N, HID), _f32),
        in_specs=[_whole((N, IN_DIM)), _whole((IN_DIM, HID)), _whole((1, HID))],
        out_specs=_whole((N, HID)),
    )(nfeat, w0t, b0r)


def _tc_msg(ef_pad, xg, w1t, b1r, w2t, b2r):
    grid = (E_PAD // T_E,)
    return pl.pallas_call(
        _msg_body,
        grid=grid,
        in_specs=[
            pl.BlockSpec((8, T_E), lambda t: (0, t)),
            pl.BlockSpec((T_E, HID), lambda t: (t, 0)),
            pl.BlockSpec((IN_DIM, 8), lambda t: (0, 0)),
            pl.BlockSpec((IN_DIM, 1), lambda t: (0, 0)),
            pl.BlockSpec((HID * HID, IN_DIM), lambda t: (0, 0)),
            pl.BlockSpec((HID * HID, 1), lambda t: (0, 0)),
        ],
        out_specs=pl.BlockSpec((T_E, HID), lambda t: (t, 0)),
        out_shape=jax.ShapeDtypeStruct((E_PAD, HID), _f32),
    )(ef_pad, xg, w1t, b1r, w2t, b2r)


def _tc_gru(aggc, degc, h, weights, biases, bconv):
    specs = ([_whole((NC, N_PAD, HID)), _whole((NC, N_PAD, 16)),
              _whole((N, HID))]
             + [_whole((HID, HID))] * 6
             + [_whole((1, HID))] * 7)
    return pl.pallas_call(
        _gru_body,
        out_shape=jax.ShapeDtypeStruct((N, HID), _f32),
        in_specs=specs,
        out_specs=_whole((N, HID)),
    )(aggc, degc, h, *weights, *biases, bconv)


def _tc_s2s(x, wq, wh, b):
    specs = ([_whole((N, HID))]
             + [_whole((2 * HID, HID))] * 4
             + [_whole((HID, HID))] * 4
             + [_whole((1, HID))] * 4)
    return pl.pallas_call(
        _s2s_body,
        out_shape=jax.ShapeDtypeStruct((1, 2 * HID), _f32),
        in_specs=specs,
        out_specs=_whole((1, 2 * HID)),
    )(x, *wq, *wh, *b)


def kernel(nfeat, efeat, edge_index, W0, b0, Wb1, bb1, Wb2, bb2, b_conv,
           gru_Wih, gru_Whh, gru_bih, gru_bhh,
           s2s_Wih, s2s_Whh, s2s_bih, s2s_bhh):
    src = edge_index[0]
    dst = edge_index[1]
    pad = E_PAD - E
    src_p = jnp.concatenate([src, jnp.zeros((pad,), jnp.int32)]
                            ).reshape(NW, KPW, CH)
    dst_p = jnp.concatenate([dst, jnp.full((pad,), N, jnp.int32)]
                            ).reshape(NW, KPW, CH)
    ef_pad = jnp.zeros((8, E_PAD), _f32).at[:5, :E].set(efeat.T)

    zeros32 = jnp.zeros((N_PAD, HID), _f32)
    zeros16 = jnp.zeros((N_PAD, 16), _f32)
    ones16 = jnp.ones((CH, 16), _f32)

    w0t = W0.T
    b0r = b0[None, :]
    w1t = jnp.zeros((IN_DIM, 8), _f32).at[:, :5].set(Wb1)
    b1r = bb1[:, None]
    w2t = Wb2.astype(jnp.bfloat16)
    b2r = bb2[:, None]
    g_w = tuple(gru_Wih[i * HID:(i + 1) * HID, :].T for i in range(3)) + \
          tuple(gru_Whh[i * HID:(i + 1) * HID, :].T for i in range(3))
    g_b = tuple(gru_bih[i * HID:(i + 1) * HID][None, :] for i in range(3)) + \
          tuple(gru_bhh[i * HID:(i + 1) * HID][None, :] for i in range(3))
    s_wq = tuple(s2s_Wih[i * HID:(i + 1) * HID, :].T for i in range(4))
    s_wh = tuple(s2s_Whh[i * HID:(i + 1) * HID, :].T for i in range(4))
    s_b = tuple((s2s_bih + s2s_bhh)[i * HID:(i + 1) * HID][None, :]
                for i in range(4))

    h = _tc_lin0(nfeat, w0t, b0r)
    degc = _sc_scatter_deg()(dst_p, ones16, zeros16)
    for _ in range(3):
        xg = _sc_gather()(src_p, h)
        msg = _tc_msg(ef_pad, xg, w1t, b1r, w2t, b2r)
        aggc = _sc_scatter_msg()(dst_p, msg, zeros32)
        h = _tc_gru(aggc, degc, h, g_w, g_b, b_conv[None, :])
    q_star = _tc_s2s(h, s_wq, s_wh, s_b)
    return (q_star, h)


# speedup vs baseline: 3.9478x; 1.0080x over previous
import functools

import jax
import jax.numpy as jnp
from jax import lax
from jax.experimental import pallas as pl
from jax.experimental.pallas import tpu as pltpu
from jax.experimental.pallas import tpu_sc as plsc

N = 10000
E = 160000
IN_DIM = 128
HID = 32

NC = 2
NS = 16
NW = NC * NS
CH = 128
KPW = 40
EPW = KPW * CH
E_PAD = NW * EPW
N_PAD = 10112
T_E = 512

_f32 = jnp.float32


@functools.cache
def _sc_mesh():
    return plsc.VectorSubcoreMesh(core_axis_name="c", subcore_axis_name="s",
                                  num_cores=NC, num_subcores=NS)


GRP = 8


def _sc_gather_body(src_hbm, h_hbm, xg_hbm, idx_v, row_v, sem):
    wid = lax.axis_index("s") * NC + lax.axis_index("c")
    base = wid * EPW
    pltpu.sync_copy(src_hbm.at[wid], idx_v)

    def body(g, carry):
        j0 = g * GRP
        cps = [pltpu.async_copy(h_hbm.at[idx_v.at[j0 + b]], row_v.at[b], sem)
               for b in range(GRP)]
        for b in range(GRP):
            cps[b].wait()
            pltpu.sync_copy(row_v.at[b],
                            xg_hbm.at[pl.ds(base + (j0 + b) * CH, CH)])
        return carry

    lax.fori_loop(0, KPW // GRP, body, 0, unroll=False)


@functools.cache
def _sc_gather():
    return pl.kernel(
        _sc_gather_body,
        out_type=jax.ShapeDtypeStruct((E_PAD, HID), _f32),
        mesh=_sc_mesh(),
        compiler_params=pltpu.CompilerParams(use_tc_tiling_on_sc=False),
        scratch_types=[
            pltpu.VMEM((KPW, CH), jnp.int32),
            pltpu.VMEM((GRP, CH, HID), _f32),
            pltpu.SemaphoreType.DMA,
        ],
    )


def _make_sc_scatter(d, per_edge_vals):
    def body(dst_hbm, vals_hbm, zero_hbm, out_hbm, idx_v, val_v, acc_s, sem,
             sem2):
        c = lax.axis_index("c")
        s = lax.axis_index("s")
        wid = s * NC + c
        base = wid * EPW
        pltpu.sync_copy(dst_hbm.at[wid], idx_v)
        if not per_edge_vals:
            for b in range(GRP):
                pltpu.sync_copy(vals_hbm, val_v.at[b])

        rpt = N_PAD // NS
        pltpu.sync_copy(zero_hbm.at[pl.ds(s * rpt, rpt)],
                        acc_s.at[pl.ds(s * rpt, rpt)])
        plsc.subcore_barrier()

        def j_body(g, carry):
            j0 = g * GRP
            if per_edge_vals:
                lds = [pltpu.async_copy(
                    vals_hbm.at[pl.ds(base + (j0 + b) * CH, CH)],
                    val_v.at[b], sem) for b in range(GRP)]
                scs = []
                for b in range(GRP):
                    lds[b].wait()
                    scs.append(pltpu.async_copy(
                        val_v.at[b], acc_s.at[idx_v.at[j0 + b]], sem2,
                        add=True))
                for cp in scs:
                    cp.wait()
            else:
                for b in range(GRP):
                    pltpu.sync_copy(val_v.at[b], acc_s.at[idx_v.at[j0 + b]],
                                    add=True)
            return carry

        lax.fori_loop(0, KPW // GRP, j_body, 0, unroll=False)
        plsc.subcore_barrier()
        pltpu.sync_copy(acc_s.at[pl.ds(s * rpt, rpt)],
                        out_hbm.at[c].at[pl.ds(s * rpt, rpt)])

    return pl.kernel(
        body,
        out_type=jax.ShapeDtypeStruct((NC, N_PAD, d), _f32),
        mesh=_sc_mesh(),
        compiler_params=pltpu.CompilerParams(use_tc_tiling_on_sc=False),
        scratch_types=[
            pltpu.VMEM((KPW, CH), jnp.int32),
            pltpu.VMEM((GRP, CH, d), _f32),
            pltpu.VMEM_SHARED((N_PAD, d), _f32),
            pltpu.SemaphoreType.DMA,
            pltpu.SemaphoreType.DMA,
        ],
    )


_sc_scatter_msg = functools.cache(lambda: _make_sc_scatter(HID, True))
_sc_scatter_deg = functools.cache(lambda: _make_sc_scatter(16, False))


def _lin0_body(x_ref, w_ref, b_ref, o_ref):
    o_ref[...] = jnp.maximum(
        jnp.dot(x_ref[...], w_ref[...], preferred_element_type=_f32)
        + b_ref[...], 0.0)


def _msg_body(ef_ref, xg_ref, w1_ref, b1_ref, w2_ref, b2_ref, o_ref):
    aT = jnp.maximum(
        jnp.dot(w1_ref[...], ef_ref[...], preferred_element_type=_f32)
        + b1_ref[...], 0.0)
    ewT = jnp.dot(w2_ref[...], aT.astype(jnp.bfloat16),
                  preferred_element_type=_f32) + b2_ref[...]
    xgT = jnp.transpose(xg_ref[...])
    acc = None
    for i in range(HID):
        term = ewT[HID * i:HID * (i + 1), :] * xgT[i:i + 1, :]
        acc = term if acc is None else acc + term
    o_ref[...] = jnp.transpose(acc)


def _gru_body(aggc_ref, degc_ref, h_ref,
              wir_ref, wiz_ref, win_ref, whr_ref, whz_ref, whn_ref,
              bir_ref, biz_ref, bin_ref, bhr_ref, bhz_ref, bhn_ref,
              bconv_ref, o_ref):
    agg = aggc_ref[0, 0:N, :] + aggc_ref[1, 0:N, :]
    deg = degc_ref[0, 0:N, 0:1] + degc_ref[1, 0:N, 0:1]
    denom = jnp.maximum(deg, 1.0)
    m = jnp.maximum(agg / denom + bconv_ref[...], 0.0)
    h = h_ref[...]

    def mm(x, w_ref):
        return jnp.dot(x, w_ref[...], preferred_element_type=_f32)

    r = jax.nn.sigmoid(mm(m, wir_ref) + bir_ref[...] + mm(h, whr_ref) + bhr_ref[...])
    z = jax.nn.sigmoid(mm(m, wiz_ref) + biz_ref[...] + mm(h, whz_ref) + bhz_ref[...])
    n = jnp.tanh(mm(m, win_ref) + bin_ref[...] + r * (mm(h, whn_ref) + bhn_ref[...]))
    o_ref[...] = (1.0 - z) * n + z * h


def _s2s_body(x_ref,
              wqi_ref, wqf_ref, wqg_ref, wqo_ref,
              whi_ref, whf_ref, whg_ref, who_ref,
              bi_ref, bf_ref, bg_ref, bo_ref, o_ref):
    x = x_ref[...]
    hl = jnp.zeros((1, HID), _f32)
    cl = jnp.zeros((1, HID), _f32)
    q = jnp.zeros((1, 2 * HID), _f32)

    def mm(v, w_ref):
        return jnp.dot(v, w_ref[...], preferred_element_type=_f32)

    for _ in range(3):
        ig = jax.nn.sigmoid(mm(q, wqi_ref) + mm(hl, whi_ref) + bi_ref[...])
        fg = jax.nn.sigmoid(mm(q, wqf_ref) + mm(hl, whf_ref) + bf_ref[...])
        gg = jnp.tanh(mm(q, wqg_ref) + mm(hl, whg_ref) + bg_ref[...])
        og = jax.nn.sigmoid(mm(q, wqo_ref) + mm(hl, who_ref) + bo_ref[...])
        cl = fg * cl + ig * gg
        hl = og * jnp.tanh(cl)
        e = jnp.sum(x * hl, axis=1, keepdims=True)
        mx = jnp.max(e, axis=0, keepdims=True)
        ex = jnp.exp(e - mx)
        alpha = ex / jnp.sum(ex, axis=0, keepdims=True)
        ro = jnp.sum(alpha * x, axis=0, keepdims=True)
        q = jnp.concatenate([hl, ro], axis=1)
    o_ref[...] = q


def _whole(shape):
    return pl.BlockSpec(shape, lambda *_: tuple(0 for _ in shape))


def _tc_lin0(nfeat, w0t, b0r):
    return pl.pallas_call(
        _lin0_body,
        out_shape=jax.ShapeDtypeStruct((N, HID), _f32),
        in_specs=[_whole((N, IN_DIM)), _whole((IN_DIM, HID)), _whole((1, HID))],
        out_specs=_whole((N, HID)),
    )(nfeat, w0t, b0r)


def _tc_msg(ef_pad, xg, w1t, b1r, w2t, b2r):
    grid = (E_PAD // T_E,)
    return pl.pallas_call(
        _msg_body,
        grid=grid,
        in_specs=[
            pl.BlockSpec((8, T_E), lambda t: (0, t)),
            pl.BlockSpec((T_E, HID), lambda t: (t, 0)),
            pl.BlockSpec((IN_DIM, 8), lambda t: (0, 0)),
            pl.BlockSpec((IN_DIM, 1), lambda t: (0, 0)),
            pl.BlockSpec((HID * HID, IN_DIM), lambda t: (0, 0)),
            pl.BlockSpec((HID * HID, 1), lambda t: (0, 0)),
        ],
        out_specs=pl.BlockSpec((T_E, HID), lambda t: (t, 0)),
        out_shape=jax.ShapeDtypeStruct((E_PAD, HID), _f32),
    )(ef_pad, xg, w1t, b1r, w2t, b2r)


def _tc_gru(aggc, degc, h, weights, biases, bconv):
    specs = ([_whole((NC, N_PAD, HID)), _whole((NC, N_PAD, 16)),
              _whole((N, HID))]
             + [_whole((HID, HID))] * 6
             + [_whole((1, HID))] * 7)
    return pl.pallas_call(
        _gru_body,
        out_shape=jax.ShapeDtypeStruct((N, HID), _f32),
        in_specs=specs,
        out_specs=_whole((N, HID)),
    )(aggc, degc, h, *weights, *biases, bconv)


def _tc_s2s(x, wq, wh, b):
    specs = ([_whole((N, HID))]
             + [_whole((2 * HID, HID))] * 4
             + [_whole((HID, HID))] * 4
             + [_whole((1, HID))] * 4)
    return pl.pallas_call(
        _s2s_body,
        out_shape=jax.ShapeDtypeStruct((1, 2 * HID), _f32),
        in_specs=specs,
        out_specs=_whole((1, 2 * HID)),
    )(x, *wq, *wh, *b)


def kernel(nfeat, efeat, edge_index, W0, b0, Wb1, bb1, Wb2, bb2, b_conv,
           gru_Wih, gru_Whh, gru_bih, gru_bhh,
           s2s_Wih, s2s_Whh, s2s_bih, s2s_bhh):
    src = edge_index[0]
    dst = edge_index[1]
    pad = E_PAD - E
    src_p = jnp.concatenate([src, jnp.zeros((pad,), jnp.int32)]
                            ).reshape(NW, KPW, CH)
    dst_p = jnp.concatenate([dst, jnp.full((pad,), N, jnp.int32)]
                            ).reshape(NW, KPW, CH)
    ef_pad = jnp.zeros((8, E_PAD), _f32).at[:5, :E].set(efeat.T)

    zeros32 = jnp.zeros((N_PAD, HID), _f32)
    zeros16 = jnp.zeros((N_PAD, 16), _f32)
    ones16 = jnp.ones((CH, 16), _f32)

    w0t = W0.T
    b0r = b0[None, :]
    w1t = jnp.zeros((IN_DIM, 8), _f32).at[:, :5].set(Wb1)
    b1r = bb1[:, None]
    w2t = Wb2.astype(jnp.bfloat16)
    b2r = bb2[:, None]
    g_w = tuple(gru_Wih[i * HID:(i + 1) * HID, :].T for i in range(3)) + \
          tuple(gru_Whh[i * HID:(i + 1) * HID, :].T for i in range(3))
    g_b = tuple(gru_bih[i * HID:(i + 1) * HID][None, :] for i in range(3)) + \
          tuple(gru_bhh[i * HID:(i + 1) * HID][None, :] for i in range(3))
    s_wq = tuple(s2s_Wih[i * HID:(i + 1) * HID, :].T for i in range(4))
    s_wh = tuple(s2s_Whh[i * HID:(i + 1) * HID, :].T for i in range(4))
    s_b = tuple((s2s_bih + s2s_bhh)[i * HID:(i + 1) * HID][None, :]
                for i in range(4))

    h = _tc_lin0(nfeat, w0t, b0r)
    degc = _sc_scatter_deg()(dst_p, ones16, zeros16)
    for _ in range(3):
        xg = _sc_gather()(src_p, h)
        msg = _tc_msg(ef_pad, xg, w1t, b1r, w2t, b2r)
        aggc = _sc_scatter_msg()(dst_p, msg, zeros32)
        h = _tc_gru(aggc, degc, h, g_w, g_b, b_conv[None, :])
    q_star = _tc_s2s(h, s_wq, s_wh, s_b)
    return (q_star, h)
